# baseline (speedup 1.0000x reference)
import jax
import jax.numpy as jnp
from jax import lax
from jax.experimental import pallas as pl
from jax.experimental.pallas import tpu as pltpu

_B, _T = 16, 20
_EMBED, _HIDDEN = 64, 256
_MEM_DIM, _HEADS, _TOPK = 64, 4, 8
_DH = _MEM_DIM // _HEADS
_C = 256
_BH = _B * _HEADS
_ROWS = _B * _MEM_DIM
_VOCAB = 100000
_VBLK = 2048
_NV = (_VOCAB + _VBLK - 1) // _VBLK


_TPAD = 24


def _tc_kernel(emb_ref, wih_e_ref, wih_r_ref, whh_ref, bl_ref, wif_ref,
               bif_ref, beta_ref, wout_ref, bout_ref, out_ref, h3_ref, hb_ref):
    v = pl.program_id(0)

    @pl.when(v == 0)
    def _recurrence():
        wih_e = wih_e_ref[...]
        wih_r = wih_r_ref[...]
        whh = whh_ref[...]
        bl = bl_ref[...]
        wif = wif_ref[...]
        bif = bif_ref[...]
        br = beta_ref[0, 0]
        bw = beta_ref[0, 1]

        g_row = lax.broadcasted_iota(jnp.int32, (64, 64), 0) // _DH
        g_col = lax.broadcasted_iota(jnp.int32, (64, 64), 1) // _DH
        G = (g_row == g_col).astype(jnp.float32)
        slot_iota = lax.broadcasted_iota(jnp.int32, (_BH, _C), 1)

        s1 = (lax.broadcasted_iota(jnp.int32, (_ROWS, _B), 0) // _MEM_DIM
              == lax.broadcasted_iota(jnp.int32, (_ROWS, _B), 1)
              ).astype(jnp.float32)
        s1t = (lax.broadcasted_iota(jnp.int32, (_B, _ROWS), 0)
               == lax.broadcasted_iota(jnp.int32, (_B, _ROWS), 1) // _MEM_DIM
               ).astype(jnp.float32)
        mask64 = (lax.broadcasted_iota(jnp.int32, (_ROWS, _MEM_DIM), 0)
                  % _MEM_DIM
                  == lax.broadcasted_iota(jnp.int32, (_ROWS, _MEM_DIM), 1)
                  ).astype(jnp.float32)
        mask256 = (lax.broadcasted_iota(jnp.int32, (_ROWS, 4 * _MEM_DIM), 0)
                   % _MEM_DIM
                   == lax.broadcasted_iota(jnp.int32, (_ROWS, 4 * _MEM_DIM), 1)
                   % _MEM_DIM).astype(jnp.float32)
        sel4 = (lax.broadcasted_iota(jnp.int32, (4 * _MEM_DIM, 4), 0)
                // _MEM_DIM
                == lax.broadcasted_iota(jnp.int32, (4 * _MEM_DIM, 4), 1)
                ).astype(jnp.float32)
        s3t = (lax.broadcasted_iota(jnp.int32, (_ROWS, _BH), 0) // _DH
               == lax.broadcasted_iota(jnp.int32, (_ROWS, _BH), 1)
               ).astype(jnp.float32)

        def normalize(k):
            gs = jnp.dot(k * k, G, preferred_element_type=jnp.float32)
            return k / (jnp.sqrt(gs) + 1e-8)

        def cols_of(x):
            m = jnp.dot(s1, x, preferred_element_type=jnp.float32)
            return jnp.dot(m * mask256, sel4,
                           preferred_element_type=jnp.float32)

        def uncol(col):
            return jnp.dot(s1t, col * mask64,
                           preferred_element_type=jnp.float32)

        def expand_rows(x):
            return jnp.dot(s3t, x, preferred_element_type=jnp.float32)

        def topk8(scores):
            s = scores
            tvs, tis = [], []
            for _ in range(_TOPK):
                m = jnp.max(s, axis=1, keepdims=True)
                sel = jnp.min(jnp.where(s == m, slot_iota, _C),
                              axis=1, keepdims=True)
                tvs.append(m)
                tis.append(sel)
                s = jnp.where(slot_iota == sel, -jnp.inf, s)
            tv = jnp.concatenate(tvs, axis=1)
            ti = jnp.concatenate(tis, axis=1)
            e = jnp.exp(tv - tv[:, 0:1])
            w = e / jnp.sum(e, axis=1, keepdims=True)
            return w, ti

        def weight_map(w, ti):
            acc = jnp.zeros((_BH, _C), jnp.float32)
            for k in range(_TOPK):
                acc = acc + jnp.where(slot_iota == ti[:, k:k + 1],
                                      w[:, k:k + 1], 0.0)
            return acc

        mem0 = jnp.full((_ROWS, _C), 1e-6, jnp.float32)
        h0 = jnp.zeros((_B, _HIDDEN), jnp.float32)
        c0 = jnp.zeros((_B, _HIDDEN), jnp.float32)
        r0 = jnp.zeros((_B, _MEM_DIM), jnp.float32)

        def step(t, carry):
            mem, h, c, rv = carry
            emb = emb_ref[t]
            gates = (jnp.dot(emb, wih_e, preferred_element_type=jnp.float32)
                     + jnp.dot(rv, wih_r, preferred_element_type=jnp.float32)
                     + jnp.dot(h, whh, preferred_element_type=jnp.float32)
                     + bl)
            i_g = jax.nn.sigmoid(gates[:, 0:256])
            f_g = jax.nn.sigmoid(gates[:, 256:512])
            g_g = jnp.tanh(gates[:, 512:768])
            o_g = jax.nn.sigmoid(gates[:, 768:1024])
            c_new = f_g * c + i_g * g_g
            h_new = o_g * jnp.tanh(c_new)
            iface = jnp.dot(h_new, wif, preferred_element_type=jnp.float32) + bif
            rk = iface[:, 0:64]
            wk = iface[:, 64:128]
            wv = iface[:, 128:192]
            er = jax.nn.sigmoid(iface[:, 192:256])

            knr = normalize(rk)
            knw = normalize(wk)
            cols = cols_of(jnp.concatenate([knr, knw, wv, er], axis=1))
            knr_col = cols[:, 0:1]
            knw_col = cols[:, 1:2]
            wv_col = cols[:, 2:3]
            er_col = cols[:, 3:4]

            sq = (mem * mem).reshape(_BH, _DH, _C)
            mem_norm = jnp.sqrt(jnp.sum(sq, axis=1))
            inv = 1.0 / (mem_norm + 1e-8)

            def scores_of(kn_col, beta):
                prod = (mem * kn_col).reshape(_BH, _DH, _C)
                return jnp.sum(prod, axis=1) * inv * beta

            rw_, ridx = topk8(scores_of(knr_col, br))
            rmap = expand_rows(weight_map(rw_, ridx))
            rvec_col = jnp.sum(mem * rmap, axis=1, keepdims=True)
            rv_new = uncol(rvec_col)

            ww_, widx = topk8(scores_of(knw_col, bw))
            wmap = expand_rows(weight_map(ww_, widx))
            mem_new = (mem * (1.0 - wmap * er_col) + wmap * wv_col)
            h3_ref[t] = h_new
            return (mem_new, h_new, c_new, rv_new)

        lax.fori_loop(0, _T, step, (mem0, h0, c0, r0))

        pr = lax.broadcasted_iota(jnp.int32, (_B * _TPAD, _T * _B), 0)
        pc = lax.broadcasted_iota(jnp.int32, (_B * _TPAD, _T * _B), 1)
        perm = (jnp.logical_and(pc == (pr % _TPAD) * _B + pr // _TPAD,
                                pr % _TPAD < _T)).astype(jnp.bfloat16)
        hmat_t = h3_ref[...].reshape(_T * _B, _HIDDEN).astype(jnp.bfloat16)
        hb_ref[...] = jnp.dot(perm, hmat_t,
                              preferred_element_type=jnp.float32
                              ).astype(jnp.bfloat16)

    m = jnp.dot(hb_ref[...], wout_ref[...].astype(jnp.bfloat16),
                preferred_element_type=jnp.float32)
    out_ref[...] = (m.reshape(_B, _TPAD, _VBLK)[:, 0:_T, :]
                    + bout_ref[...])


def _gather_embeddings(embedding, idx):
    return jnp.take(embedding, idx, axis=0)


def kernel(input_seq, embedding, W_ih, W_hh, b_lstm, W_iface, b_iface,
           W_out, b_out, beta_read, beta_write):
    idx = input_seq.T.reshape(-1)
    emb_all = _gather_embeddings(embedding, idx).reshape(_T, _B, _EMBED)
    br = jnp.abs(beta_read) + 1e-6
    bw = jnp.abs(beta_write) + 1e-6
    betas = jnp.stack([br, bw]).reshape(1, 2).astype(jnp.float32)

    out = pl.pallas_call(
        _tc_kernel,
        grid=(_NV,),
        in_specs=[
            pl.BlockSpec((_T, _B, _EMBED), lambda v: (0, 0, 0)),
            pl.BlockSpec((_EMBED, 4 * _HIDDEN), lambda v: (0, 0)),
            pl.BlockSpec((_MEM_DIM, 4 * _HIDDEN), lambda v: (0, 0)),
            pl.BlockSpec((_HIDDEN, 4 * _HIDDEN), lambda v: (0, 0)),
            pl.BlockSpec((1, 4 * _HIDDEN), lambda v: (0, 0)),
            pl.BlockSpec((_HIDDEN, _HIDDEN), lambda v: (0, 0)),
            pl.BlockSpec((1, _HIDDEN), lambda v: (0, 0)),
            pl.BlockSpec(memory_space=pltpu.SMEM),
            pl.BlockSpec((_HIDDEN, _VBLK), lambda v: (0, v)),
            pl.BlockSpec((1, _VBLK), lambda v: (0, v)),
        ],
        out_specs=pl.BlockSpec((_B, _T, _VBLK), lambda v: (0, 0, v)),
        out_shape=jax.ShapeDtypeStruct((_B, _T, _VOCAB), jnp.float32),
        scratch_shapes=[pltpu.VMEM((_T, _B, _HIDDEN), jnp.float32),
                        pltpu.VMEM((_B * _TPAD, _HIDDEN), jnp.bfloat16)],
    )(emb_all, W_ih[:_EMBED], W_ih[_EMBED:], W_hh, b_lstm.reshape(1, -1),
      W_iface, b_iface.reshape(1, -1), betas, W_out, b_out.reshape(1, -1))

    return out


# speedup vs baseline: 1.3087x; 1.3087x over previous
import jax
import jax.numpy as jnp
from jax import lax
from jax.experimental import pallas as pl
from jax.experimental.pallas import tpu as pltpu

_B, _T = 16, 20
_EMBED, _HIDDEN = 64, 256
_MEM_DIM, _HEADS, _TOPK = 64, 4, 8
_DH = _MEM_DIM // _HEADS
_C = 256
_BH = _B * _HEADS
_ROWS = _B * _MEM_DIM
_VOCAB = 100000
_VBLK = 4096
_NV = (_VOCAB + _VBLK - 1) // _VBLK


def _tc_kernel(emb_ref, wih_e_ref, wih_r_ref, whh_ref, bl_ref, wif_ref,
               bif_ref, beta_ref, wout_ref, bout_ref, out_ref, h3_ref):
    v = pl.program_id(0)

    @pl.when(v == 0)
    def _recurrence():
        wih_e = wih_e_ref[...]
        wih_r = wih_r_ref[...]
        whh = whh_ref[...]
        bl = bl_ref[...]
        wif = wif_ref[...]
        bif = bif_ref[...]
        br = beta_ref[0, 0]
        bw = beta_ref[0, 1]

        g_row = lax.broadcasted_iota(jnp.int32, (64, 64), 0) // _DH
        g_col = lax.broadcasted_iota(jnp.int32, (64, 64), 1) // _DH
        G = (g_row == g_col).astype(jnp.float32)
        slot_iota = lax.broadcasted_iota(jnp.int32, (_BH, _C), 1)

        s1 = (lax.broadcasted_iota(jnp.int32, (_ROWS, _B), 0) // _MEM_DIM
              == lax.broadcasted_iota(jnp.int32, (_ROWS, _B), 1)
              ).astype(jnp.float32)
        s1t = (lax.broadcasted_iota(jnp.int32, (_B, _ROWS), 0)
               == lax.broadcasted_iota(jnp.int32, (_B, _ROWS), 1) // _MEM_DIM
               ).astype(jnp.float32)
        mask64 = (lax.broadcasted_iota(jnp.int32, (_ROWS, _MEM_DIM), 0)
                  % _MEM_DIM
                  == lax.broadcasted_iota(jnp.int32, (_ROWS, _MEM_DIM), 1)
                  ).astype(jnp.float32)
        mask256 = (lax.broadcasted_iota(jnp.int32, (_ROWS, 4 * _MEM_DIM), 0)
                   % _MEM_DIM
                   == lax.broadcasted_iota(jnp.int32, (_ROWS, 4 * _MEM_DIM), 1)
                   % _MEM_DIM).astype(jnp.float32)
        sel4 = (lax.broadcasted_iota(jnp.int32, (4 * _MEM_DIM, 4), 0)
                // _MEM_DIM
                == lax.broadcasted_iota(jnp.int32, (4 * _MEM_DIM, 4), 1)
                ).astype(jnp.float32)
        s3t = (lax.broadcasted_iota(jnp.int32, (_ROWS, _BH), 0) // _DH
               == lax.broadcasted_iota(jnp.int32, (_ROWS, _BH), 1)
               ).astype(jnp.float32)

        def normalize(k):
            gs = jnp.dot(k * k, G, preferred_element_type=jnp.float32)
            return k / (jnp.sqrt(gs) + 1e-8)

        def cols_of(x):
            m = jnp.dot(s1, x, preferred_element_type=jnp.float32)
            return jnp.dot(m * mask256, sel4,
                           preferred_element_type=jnp.float32)

        def uncol(col):
            return jnp.dot(s1t, col * mask64,
                           preferred_element_type=jnp.float32)

        def expand_rows(x):
            return jnp.dot(s3t, x, preferred_element_type=jnp.float32)

        def topk8(scores):
            s = scores
            tvs, tis = [], []
            for _ in range(_TOPK):
                m = jnp.max(s, axis=1, keepdims=True)
                sel = jnp.min(jnp.where(s == m, slot_iota, _C),
                              axis=1, keepdims=True)
                tvs.append(m)
                tis.append(sel)
                s = jnp.where(slot_iota == sel, -jnp.inf, s)
            tv = jnp.concatenate(tvs, axis=1)
            ti = jnp.concatenate(tis, axis=1)
            e = jnp.exp(tv - tv[:, 0:1])
            w = e / jnp.sum(e, axis=1, keepdims=True)
            return w, ti

        def weight_map(w, ti):
            acc = jnp.zeros((_BH, _C), jnp.float32)
            for k in range(_TOPK):
                acc = acc + jnp.where(slot_iota == ti[:, k:k + 1],
                                      w[:, k:k + 1], 0.0)
            return acc

        mem0 = jnp.full((_ROWS, _C), 1e-6, jnp.float32)
        h0 = jnp.zeros((_B, _HIDDEN), jnp.float32)
        c0 = jnp.zeros((_B, _HIDDEN), jnp.float32)
        r0 = jnp.zeros((_B, _MEM_DIM), jnp.float32)

        def step(t, carry):
            mem, h, c, rv = carry
            emb = emb_ref[t]
            gates = (jnp.dot(emb, wih_e, preferred_element_type=jnp.float32)
                     + jnp.dot(rv, wih_r, preferred_element_type=jnp.float32)
                     + jnp.dot(h, whh, preferred_element_type=jnp.float32)
                     + bl)
            i_g = jax.nn.sigmoid(gates[:, 0:256])
            f_g = jax.nn.sigmoid(gates[:, 256:512])
            g_g = jnp.tanh(gates[:, 512:768])
            o_g = jax.nn.sigmoid(gates[:, 768:1024])
            c_new = f_g * c + i_g * g_g
            h_new = o_g * jnp.tanh(c_new)
            iface = jnp.dot(h_new, wif, preferred_element_type=jnp.float32) + bif
            rk = iface[:, 0:64]
            wk = iface[:, 64:128]
            wv = iface[:, 128:192]
            er = jax.nn.sigmoid(iface[:, 192:256])

            knr = normalize(rk)
            knw = normalize(wk)
            cols = cols_of(jnp.concatenate([knr, knw, wv, er], axis=1))
            knr_col = cols[:, 0:1]
            knw_col = cols[:, 1:2]
            wv_col = cols[:, 2:3]
            er_col = cols[:, 3:4]

            sq = (mem * mem).reshape(_BH, _DH, _C)
            mem_norm = jnp.sqrt(jnp.sum(sq, axis=1))
            inv = 1.0 / (mem_norm + 1e-8)

            def scores_of(kn_col, beta):
                prod = (mem * kn_col).reshape(_BH, _DH, _C)
                return jnp.sum(prod, axis=1) * inv * beta

            rw_, ridx = topk8(scores_of(knr_col, br))
            rmap = expand_rows(weight_map(rw_, ridx))
            rvec_col = jnp.sum(mem * rmap, axis=1, keepdims=True)
            rv_new = uncol(rvec_col)

            ww_, widx = topk8(scores_of(knw_col, bw))
            wmap = expand_rows(weight_map(ww_, widx))
            mem_new = (mem * (1.0 - wmap * er_col) + wmap * wv_col)
            h3_ref[t] = h_new
            return (mem_new, h_new, c_new, rv_new)

        lax.fori_loop(0, _T, step, (mem0, h0, c0, r0))

    hmat = h3_ref[...].reshape(_T * _B, _HIDDEN)
    out_ref[...] = (jnp.dot(hmat, wout_ref[...],
                            preferred_element_type=jnp.float32)
                    + bout_ref[...])


def _gather_embeddings(embedding, idx):
    return jnp.take(embedding, idx, axis=0)


def kernel(input_seq, embedding, W_ih, W_hh, b_lstm, W_iface, b_iface,
           W_out, b_out, beta_read, beta_write):
    idx = input_seq.T.reshape(-1)
    emb_all = _gather_embeddings(embedding, idx).reshape(_T, _B, _EMBED)
    br = jnp.abs(beta_read) + 1e-6
    bw = jnp.abs(beta_write) + 1e-6
    betas = jnp.stack([br, bw]).reshape(1, 2).astype(jnp.float32)

    out = pl.pallas_call(
        _tc_kernel,
        grid=(_NV,),
        in_specs=[
            pl.BlockSpec((_T, _B, _EMBED), lambda v: (0, 0, 0)),
            pl.BlockSpec((_EMBED, 4 * _HIDDEN), lambda v: (0, 0)),
            pl.BlockSpec((_MEM_DIM, 4 * _HIDDEN), lambda v: (0, 0)),
            pl.BlockSpec((_HIDDEN, 4 * _HIDDEN), lambda v: (0, 0)),
            pl.BlockSpec((1, 4 * _HIDDEN), lambda v: (0, 0)),
            pl.BlockSpec((_HIDDEN, _HIDDEN), lambda v: (0, 0)),
            pl.BlockSpec((1, _HIDDEN), lambda v: (0, 0)),
            pl.BlockSpec(memory_space=pltpu.SMEM),
            pl.BlockSpec((_HIDDEN, _VBLK), lambda v: (0, v)),
            pl.BlockSpec((1, _VBLK), lambda v: (0, v)),
        ],
        out_specs=pl.BlockSpec((_T * _B, _VBLK), lambda v: (0, v)),
        out_shape=jax.ShapeDtypeStruct((_T * _B, _VOCAB), jnp.float32),
        scratch_shapes=[pltpu.VMEM((_T, _B, _HIDDEN), jnp.float32)],
    )(emb_all, W_ih[:_EMBED], W_ih[_EMBED:], W_hh, b_lstm.reshape(1, -1),
      W_iface, b_iface.reshape(1, -1), betas, W_out, b_out.reshape(1, -1))

    return out.reshape(_T, _B, _VOCAB).transpose(1, 0, 2)


# speedup vs baseline: 1.3175x; 1.0067x over previous
import jax
import jax.numpy as jnp
from jax import lax
from jax.experimental import pallas as pl
from jax.experimental.pallas import tpu as pltpu

_B, _T = 16, 20
_EMBED, _HIDDEN = 64, 256
_MEM_DIM, _HEADS, _TOPK = 64, 4, 8
_DH = _MEM_DIM // _HEADS
_C = 256
_BH = _B * _HEADS
_ROWS = _B * _MEM_DIM
_VOCAB = 100000
_VBLK = 8192
_NV = (_VOCAB + _VBLK - 1) // _VBLK


def _tc_kernel(emb_ref, wih_e_ref, wih_r_ref, whh_ref, bl_ref, wif_ref,
               bif_ref, beta_ref, wout_ref, bout_ref, out_ref, h3_ref):
    v = pl.program_id(0)

    @pl.when(v == 0)
    def _recurrence():
        wih_e = wih_e_ref[...]
        wih_r = wih_r_ref[...]
        whh = whh_ref[...]
        bl = bl_ref[...]
        wif = wif_ref[...]
        bif = bif_ref[...]
        br = beta_ref[0, 0]
        bw = beta_ref[0, 1]

        g_row = lax.broadcasted_iota(jnp.int32, (64, 64), 0) // _DH
        g_col = lax.broadcasted_iota(jnp.int32, (64, 64), 1) // _DH
        G = (g_row == g_col).astype(jnp.float32)
        slot_iota = lax.broadcasted_iota(jnp.int32, (_BH, _C), 1)

        s1 = (lax.broadcasted_iota(jnp.int32, (_ROWS, _B), 0) // _MEM_DIM
              == lax.broadcasted_iota(jnp.int32, (_ROWS, _B), 1)
              ).astype(jnp.float32)
        s1t = (lax.broadcasted_iota(jnp.int32, (_B, _ROWS), 0)
               == lax.broadcasted_iota(jnp.int32, (_B, _ROWS), 1) // _MEM_DIM
               ).astype(jnp.float32)
        mask64 = (lax.broadcasted_iota(jnp.int32, (_ROWS, _MEM_DIM), 0)
                  % _MEM_DIM
                  == lax.broadcasted_iota(jnp.int32, (_ROWS, _MEM_DIM), 1)
                  ).astype(jnp.float32)
        mask256 = (lax.broadcasted_iota(jnp.int32, (_ROWS, 4 * _MEM_DIM), 0)
                   % _MEM_DIM
                   == lax.broadcasted_iota(jnp.int32, (_ROWS, 4 * _MEM_DIM), 1)
                   % _MEM_DIM).astype(jnp.float32)
        sel4 = (lax.broadcasted_iota(jnp.int32, (4 * _MEM_DIM, 4), 0)
                // _MEM_DIM
                == lax.broadcasted_iota(jnp.int32, (4 * _MEM_DIM, 4), 1)
                ).astype(jnp.float32)
        s3t = (lax.broadcasted_iota(jnp.int32, (_ROWS, _BH), 0) // _DH
               == lax.broadcasted_iota(jnp.int32, (_ROWS, _BH), 1)
               ).astype(jnp.float32)

        def normalize(k):
            gs = jnp.dot(k * k, G, preferred_element_type=jnp.float32)
            return k / (jnp.sqrt(gs) + 1e-8)

        def cols_of(x):
            m = jnp.dot(s1, x, preferred_element_type=jnp.float32)
            return jnp.dot(m * mask256, sel4,
                           preferred_element_type=jnp.float32)

        def uncol(col):
            return jnp.dot(s1t, col * mask64,
                           preferred_element_type=jnp.float32)

        def expand_rows(x):
            return jnp.dot(s3t, x, preferred_element_type=jnp.float32)

        def topk8(scores):
            s = scores
            tvs, tis = [], []
            for _ in range(_TOPK):
                m = jnp.max(s, axis=1, keepdims=True)
                sel = jnp.min(jnp.where(s == m, slot_iota, _C),
                              axis=1, keepdims=True)
                tvs.append(m)
                tis.append(sel)
                s = jnp.where(slot_iota == sel, -jnp.inf, s)
            tv = jnp.concatenate(tvs, axis=1)
            ti = jnp.concatenate(tis, axis=1)
            e = jnp.exp(tv - tv[:, 0:1])
            w = e / jnp.sum(e, axis=1, keepdims=True)
            return w, ti

        def weight_map(w, ti):
            acc = jnp.zeros((_BH, _C), jnp.float32)
            for k in range(_TOPK):
                acc = acc + jnp.where(slot_iota == ti[:, k:k + 1],
                                      w[:, k:k + 1], 0.0)
            return acc

        mem0 = jnp.full((_ROWS, _C), 1e-6, jnp.float32)
        h0 = jnp.zeros((_B, _HIDDEN), jnp.float32)
        c0 = jnp.zeros((_B, _HIDDEN), jnp.float32)
        r0 = jnp.zeros((_B, _MEM_DIM), jnp.float32)

        def step(t, carry):
            mem, h, c, rv = carry
            emb = emb_ref[t]
            gates = (jnp.dot(emb, wih_e, preferred_element_type=jnp.float32)
                     + jnp.dot(rv, wih_r, preferred_element_type=jnp.float32)
                     + jnp.dot(h, whh, preferred_element_type=jnp.float32)
                     + bl)
            i_g = jax.nn.sigmoid(gates[:, 0:256])
            f_g = jax.nn.sigmoid(gates[:, 256:512])
            g_g = jnp.tanh(gates[:, 512:768])
            o_g = jax.nn.sigmoid(gates[:, 768:1024])
            c_new = f_g * c + i_g * g_g
            h_new = o_g * jnp.tanh(c_new)
            iface = jnp.dot(h_new, wif, preferred_element_type=jnp.float32) + bif
            rk = iface[:, 0:64]
            wk = iface[:, 64:128]
            wv = iface[:, 128:192]
            er = jax.nn.sigmoid(iface[:, 192:256])

            knr = normalize(rk)
            knw = normalize(wk)
            cols = cols_of(jnp.concatenate([knr, knw, wv, er], axis=1))
            knr_col = cols[:, 0:1]
            knw_col = cols[:, 1:2]
            wv_col = cols[:, 2:3]
            er_col = cols[:, 3:4]

            sq = (mem * mem).reshape(_BH, _DH, _C)
            mem_norm = jnp.sqrt(jnp.sum(sq, axis=1))
            inv = 1.0 / (mem_norm + 1e-8)

            def scores_of(kn_col, beta):
                prod = (mem * kn_col).reshape(_BH, _DH, _C)
                return jnp.sum(prod, axis=1) * inv * beta

            rw_, ridx = topk8(scores_of(knr_col, br))
            rmap = expand_rows(weight_map(rw_, ridx))
            rvec_col = jnp.sum(mem * rmap, axis=1, keepdims=True)
            rv_new = uncol(rvec_col)

            ww_, widx = topk8(scores_of(knw_col, bw))
            wmap = expand_rows(weight_map(ww_, widx))
            mem_new = (mem * (1.0 - wmap * er_col) + wmap * wv_col)
            h3_ref[t] = h_new
            return (mem_new, h_new, c_new, rv_new)

        lax.fori_loop(0, _T, step, (mem0, h0, c0, r0))

    hmat = h3_ref[...].reshape(_T * _B, _HIDDEN)
    out_ref[...] = (jnp.dot(hmat, wout_ref[...],
                            preferred_element_type=jnp.float32)
                    + bout_ref[...])


def _gather_embeddings(embedding, idx):
    return jnp.take(embedding, idx, axis=0)


def kernel(input_seq, embedding, W_ih, W_hh, b_lstm, W_iface, b_iface,
           W_out, b_out, beta_read, beta_write):
    idx = input_seq.T.reshape(-1)
    emb_all = _gather_embeddings(embedding, idx).reshape(_T, _B, _EMBED)
    br = jnp.abs(beta_read) + 1e-6
    bw = jnp.abs(beta_write) + 1e-6
    betas = jnp.stack([br, bw]).reshape(1, 2).astype(jnp.float32)

    out = pl.pallas_call(
        _tc_kernel,
        grid=(_NV,),
        in_specs=[
            pl.BlockSpec((_T, _B, _EMBED), lambda v: (0, 0, 0)),
            pl.BlockSpec((_EMBED, 4 * _HIDDEN), lambda v: (0, 0)),
            pl.BlockSpec((_MEM_DIM, 4 * _HIDDEN), lambda v: (0, 0)),
            pl.BlockSpec((_HIDDEN, 4 * _HIDDEN), lambda v: (0, 0)),
            pl.BlockSpec((1, 4 * _HIDDEN), lambda v: (0, 0)),
            pl.BlockSpec((_HIDDEN, _HIDDEN), lambda v: (0, 0)),
            pl.BlockSpec((1, _HIDDEN), lambda v: (0, 0)),
            pl.BlockSpec(memory_space=pltpu.SMEM),
            pl.BlockSpec((_HIDDEN, _VBLK), lambda v: (0, v)),
            pl.BlockSpec((1, _VBLK), lambda v: (0, v)),
        ],
        out_specs=pl.BlockSpec((_T * _B, _VBLK), lambda v: (0, v)),
        out_shape=jax.ShapeDtypeStruct((_T * _B, _VOCAB), jnp.float32),
        scratch_shapes=[pltpu.VMEM((_T, _B, _HIDDEN), jnp.float32)],
    )(emb_all, W_ih[:_EMBED], W_ih[_EMBED:], W_hh, b_lstm.reshape(1, -1),
      W_iface, b_iface.reshape(1, -1), betas, W_out, b_out.reshape(1, -1))

    return out.reshape(_T, _B, _VOCAB).transpose(1, 0, 2)


# speedup vs baseline: 1.3182x; 1.0005x over previous
import jax
import jax.numpy as jnp
from jax import lax
from jax.experimental import pallas as pl
from jax.experimental.pallas import tpu as pltpu

_B, _T = 16, 20
_EMBED, _HIDDEN = 64, 256
_MEM_DIM, _HEADS, _TOPK = 64, 4, 8
_DH = _MEM_DIM // _HEADS
_C = 256
_BH = _B * _HEADS
_ROWS = _B * _MEM_DIM
_VOCAB = 100000
_VBLK = 8192
_NV = (_VOCAB + _VBLK - 1) // _VBLK


def _tc_kernel(emb_ref, wih_e_ref, wih_r_ref, whh_ref, bl_ref, wif_ref,
               bif_ref, beta_ref, wout_ref, bout_ref, out_ref, h3_ref):
    v = pl.program_id(0)

    @pl.when(v == 0)
    def _recurrence():
        wih_e = wih_e_ref[...]
        wih_r = wih_r_ref[...]
        whh = whh_ref[...]
        bl = bl_ref[...]
        wif = wif_ref[...]
        bif = bif_ref[...]
        br = beta_ref[0, 0]
        bw = beta_ref[0, 1]

        g_row = lax.broadcasted_iota(jnp.int32, (64, 64), 0) // _DH
        g_col = lax.broadcasted_iota(jnp.int32, (64, 64), 1) // _DH
        G = (g_row == g_col).astype(jnp.float32)
        slot_iota = lax.broadcasted_iota(jnp.int32, (_BH, _C), 1)

        s1 = (lax.broadcasted_iota(jnp.int32, (_ROWS, _B), 0) // _MEM_DIM
              == lax.broadcasted_iota(jnp.int32, (_ROWS, _B), 1)
              ).astype(jnp.float32)
        s1t = (lax.broadcasted_iota(jnp.int32, (_B, _ROWS), 0)
               == lax.broadcasted_iota(jnp.int32, (_B, _ROWS), 1) // _MEM_DIM
               ).astype(jnp.float32)
        mask64 = (lax.broadcasted_iota(jnp.int32, (_ROWS, _MEM_DIM), 0)
                  % _MEM_DIM
                  == lax.broadcasted_iota(jnp.int32, (_ROWS, _MEM_DIM), 1)
                  ).astype(jnp.float32)
        mask256 = (lax.broadcasted_iota(jnp.int32, (_ROWS, 4 * _MEM_DIM), 0)
                   % _MEM_DIM
                   == lax.broadcasted_iota(jnp.int32, (_ROWS, 4 * _MEM_DIM), 1)
                   % _MEM_DIM).astype(jnp.float32)
        sel4 = (lax.broadcasted_iota(jnp.int32, (4 * _MEM_DIM, 4), 0)
                // _MEM_DIM
                == lax.broadcasted_iota(jnp.int32, (4 * _MEM_DIM, 4), 1)
                ).astype(jnp.float32)
        s3t = (lax.broadcasted_iota(jnp.int32, (_ROWS, _BH), 0) // _DH
               == lax.broadcasted_iota(jnp.int32, (_ROWS, _BH), 1)
               ).astype(jnp.float32)

        def normalize(k):
            gs = jnp.dot(k * k, G, preferred_element_type=jnp.float32)
            return k / (jnp.sqrt(gs) + 1e-8)

        def cols_of(x):
            m = jnp.dot(s1, x, preferred_element_type=jnp.float32)
            return jnp.dot(m * mask256, sel4,
                           preferred_element_type=jnp.float32)

        def uncol(col):
            return jnp.dot(s1t, col * mask64,
                           preferred_element_type=jnp.float32)

        def expand_rows(x):
            return jnp.dot(s3t, x, preferred_element_type=jnp.float32)

        def topk8(scores):
            s = scores
            tvs, tis = [], []
            for _ in range(_TOPK):
                m = jnp.max(s, axis=1, keepdims=True)
                sel = jnp.min(jnp.where(s == m, slot_iota, _C),
                              axis=1, keepdims=True)
                tvs.append(m)
                tis.append(sel)
                s = jnp.where(slot_iota == sel, -jnp.inf, s)
            tv = jnp.concatenate(tvs, axis=1)
            ti = jnp.concatenate(tis, axis=1)
            e = jnp.exp(tv - tv[:, 0:1])
            w = e / jnp.sum(e, axis=1, keepdims=True)
            return w, ti

        def weight_map(w, ti):
            acc = jnp.zeros((_BH, _C), jnp.float32)
            for k in range(_TOPK):
                acc = acc + jnp.where(slot_iota == ti[:, k:k + 1],
                                      w[:, k:k + 1], 0.0)
            return acc

        mem0 = jnp.full((_ROWS, _C), 1e-6, jnp.float32)
        h0 = jnp.zeros((_B, _HIDDEN), jnp.float32)
        c0 = jnp.zeros((_B, _HIDDEN), jnp.float32)
        r0 = jnp.zeros((_B, _MEM_DIM), jnp.float32)

        def step(t, carry):
            mem, h, c, rv = carry
            emb = emb_ref[t]
            gates = (jnp.dot(emb, wih_e, preferred_element_type=jnp.float32)
                     + jnp.dot(rv, wih_r, preferred_element_type=jnp.float32)
                     + jnp.dot(h, whh, preferred_element_type=jnp.float32)
                     + bl)
            i_g = jax.nn.sigmoid(gates[:, 0:256])
            f_g = jax.nn.sigmoid(gates[:, 256:512])
            g_g = jnp.tanh(gates[:, 512:768])
            o_g = jax.nn.sigmoid(gates[:, 768:1024])
            c_new = f_g * c + i_g * g_g
            h_new = o_g * jnp.tanh(c_new)
            iface = jnp.dot(h_new, wif, preferred_element_type=jnp.float32) + bif
            rk = iface[:, 0:64]
            wk = iface[:, 64:128]
            wv = iface[:, 128:192]
            er = jax.nn.sigmoid(iface[:, 192:256])

            knr = normalize(rk)
            knw = normalize(wk)
            cols = cols_of(jnp.concatenate([knr, knw, wv, er], axis=1))
            knr_col = cols[:, 0:1]
            knw_col = cols[:, 1:2]
            wv_col = cols[:, 2:3]
            er_col = cols[:, 3:4]

            sq = (mem * mem).reshape(_BH, _DH, _C)
            mem_norm = jnp.sqrt(jnp.sum(sq, axis=1))
            inv = 1.0 / (mem_norm + 1e-8)

            def scores_of(kn_col, beta):
                prod = (mem * kn_col).reshape(_BH, _DH, _C)
                return jnp.sum(prod, axis=1) * inv * beta

            rw_, ridx = topk8(scores_of(knr_col, br))
            rmap = expand_rows(weight_map(rw_, ridx))
            rvec_col = jnp.sum(mem * rmap, axis=1, keepdims=True)
            rv_new = uncol(rvec_col)

            ww_, widx = topk8(scores_of(knw_col, bw))
            wmap = expand_rows(weight_map(ww_, widx))
            mem_new = (mem * (1.0 - wmap * er_col) + wmap * wv_col)
            h3_ref[t] = h_new
            return (mem_new, h_new, c_new, rv_new)

        lax.fori_loop(0, _T, step, (mem0, h0, c0, r0))

    hmat = h3_ref[...].reshape(_T * _B, _HIDDEN).astype(jnp.bfloat16)
    out_ref[...] = (jnp.dot(hmat, wout_ref[...].astype(jnp.bfloat16),
                            preferred_element_type=jnp.float32)
                    + bout_ref[...])


def _gather_embeddings(embedding, idx):
    return jnp.take(embedding, idx, axis=0)


def kernel(input_seq, embedding, W_ih, W_hh, b_lstm, W_iface, b_iface,
           W_out, b_out, beta_read, beta_write):
    idx = input_seq.T.reshape(-1)
    emb_all = _gather_embeddings(embedding, idx).reshape(_T, _B, _EMBED)
    br = jnp.abs(beta_read) + 1e-6
    bw = jnp.abs(beta_write) + 1e-6
    betas = jnp.stack([br, bw]).reshape(1, 2).astype(jnp.float32)

    out = pl.pallas_call(
        _tc_kernel,
        grid=(_NV,),
        in_specs=[
            pl.BlockSpec((_T, _B, _EMBED), lambda v: (0, 0, 0)),
            pl.BlockSpec((_EMBED, 4 * _HIDDEN), lambda v: (0, 0)),
            pl.BlockSpec((_MEM_DIM, 4 * _HIDDEN), lambda v: (0, 0)),
            pl.BlockSpec((_HIDDEN, 4 * _HIDDEN), lambda v: (0, 0)),
            pl.BlockSpec((1, 4 * _HIDDEN), lambda v: (0, 0)),
            pl.BlockSpec((_HIDDEN, _HIDDEN), lambda v: (0, 0)),
            pl.BlockSpec((1, _HIDDEN), lambda v: (0, 0)),
            pl.BlockSpec(memory_space=pltpu.SMEM),
            pl.BlockSpec((_HIDDEN, _VBLK), lambda v: (0, v)),
            pl.BlockSpec((1, _VBLK), lambda v: (0, v)),
        ],
        out_specs=pl.BlockSpec((_T * _B, _VBLK), lambda v: (0, v)),
        out_shape=jax.ShapeDtypeStruct((_T * _B, _VOCAB), jnp.float32),
        scratch_shapes=[pltpu.VMEM((_T, _B, _HIDDEN), jnp.float32)],
    )(emb_all, W_ih[:_EMBED], W_ih[_EMBED:], W_hh, b_lstm.reshape(1, -1),
      W_iface, b_iface.reshape(1, -1), betas, W_out, b_out.reshape(1, -1))

    return out.reshape(_T, _B, _VOCAB).transpose(1, 0, 2)


# speedup vs baseline: 1.3994x; 1.0617x over previous
import jax
import jax.numpy as jnp
from jax import lax
from jax.experimental import pallas as pl
from jax.experimental.pallas import tpu as pltpu

_B, _T = 16, 20
_EMBED, _HIDDEN = 64, 256
_MEM_DIM, _HEADS, _TOPK = 64, 4, 8
_DH = _MEM_DIM // _HEADS
_C = 256
_BH = _B * _HEADS
_ROWS = _B * _MEM_DIM
_VOCAB = 100000
_VBLK = 8192
_NV = (_VOCAB + _VBLK - 1) // _VBLK


def _tc_kernel(emb_ref, wih_e_ref, wih_r_ref, whh_ref, bl_ref, wif_ref,
               bif_ref, beta_ref, wout_ref, bout_ref, out_ref, h3_ref,
               mem_ref):
    v = pl.program_id(0)

    @pl.when(v == 0)
    def _recurrence():
        wih_e = wih_e_ref[...]
        wih_r = wih_r_ref[...]
        whh = whh_ref[...]
        bl = bl_ref[...]
        wif = wif_ref[...]
        bif = bif_ref[...]
        br = beta_ref[0, 0]
        bw = beta_ref[0, 1]

        g_row = lax.broadcasted_iota(jnp.int32, (64, 64), 0) // _DH
        g_col = lax.broadcasted_iota(jnp.int32, (64, 64), 1) // _DH
        G = (g_row == g_col).astype(jnp.float32)
        slot_iota = lax.broadcasted_iota(jnp.int32, (_BH, _C), 1)

        s1 = (lax.broadcasted_iota(jnp.int32, (_ROWS, _B), 0) // _MEM_DIM
              == lax.broadcasted_iota(jnp.int32, (_ROWS, _B), 1)
              ).astype(jnp.float32)
        s1t = (lax.broadcasted_iota(jnp.int32, (_B, _ROWS), 0)
               == lax.broadcasted_iota(jnp.int32, (_B, _ROWS), 1) // _MEM_DIM
               ).astype(jnp.float32)
        mask64 = (lax.broadcasted_iota(jnp.int32, (_ROWS, _MEM_DIM), 0)
                  % _MEM_DIM
                  == lax.broadcasted_iota(jnp.int32, (_ROWS, _MEM_DIM), 1)
                  ).astype(jnp.float32)
        mask256 = (lax.broadcasted_iota(jnp.int32, (_ROWS, 4 * _MEM_DIM), 0)
                   % _MEM_DIM
                   == lax.broadcasted_iota(jnp.int32, (_ROWS, 4 * _MEM_DIM), 1)
                   % _MEM_DIM).astype(jnp.float32)
        sel4 = (lax.broadcasted_iota(jnp.int32, (4 * _MEM_DIM, 4), 0)
                // _MEM_DIM
                == lax.broadcasted_iota(jnp.int32, (4 * _MEM_DIM, 4), 1)
                ).astype(jnp.float32)
        s3t = (lax.broadcasted_iota(jnp.int32, (_ROWS, _BH), 0) // _DH
               == lax.broadcasted_iota(jnp.int32, (_ROWS, _BH), 1)
               ).astype(jnp.float32)
        s3 = (lax.broadcasted_iota(jnp.int32, (_BH, _ROWS), 0)
              == lax.broadcasted_iota(jnp.int32, (_BH, _ROWS), 1) // _DH
              ).astype(jnp.float32)

        def normalize(k):
            gs = jnp.dot(k * k, G, preferred_element_type=jnp.float32)
            return k / (jnp.sqrt(gs) + 1e-8)

        def cols_of(x):
            m = jnp.dot(s1, x, preferred_element_type=jnp.float32)
            return jnp.dot(m * mask256, sel4,
                           preferred_element_type=jnp.float32)

        def uncol(col):
            return jnp.dot(s1t, col * mask64,
                           preferred_element_type=jnp.float32)

        def expand_rows(x):
            return jnp.dot(s3t, x, preferred_element_type=jnp.float32)

        slot_iota2 = lax.broadcasted_iota(jnp.int32, (2 * _BH, _C), 1)

        def topk8(scores):
            s = scores
            tvs, tis = [], []
            for _ in range(_TOPK):
                m = jnp.max(s, axis=1, keepdims=True)
                sel = jnp.min(jnp.where(s == m, slot_iota2, _C),
                              axis=1, keepdims=True)
                tvs.append(m)
                tis.append(sel)
                s = jnp.where(slot_iota2 == sel, -jnp.inf, s)
            tv = jnp.concatenate(tvs, axis=1)
            ti = jnp.concatenate(tis, axis=1)
            e = jnp.exp(tv - tv[:, 0:1])
            w = e / jnp.sum(e, axis=1, keepdims=True)
            return w, ti

        def weight_map(w, ti):
            acc = jnp.zeros((2 * _BH, _C), jnp.float32)
            for k in range(_TOPK):
                acc = acc + jnp.where(slot_iota2 == ti[:, k:k + 1],
                                      w[:, k:k + 1], 0.0)
            return acc

        mem_ref[...] = jnp.full((_ROWS, _C), 1e-6, jnp.float32)
        h0 = jnp.zeros((_B, _HIDDEN), jnp.float32)
        c0 = jnp.zeros((_B, _HIDDEN), jnp.float32)
        r0 = jnp.zeros((_B, _MEM_DIM), jnp.float32)

        def step(t, carry):
            h, c, rv = carry
            mem = mem_ref[...]
            emb = emb_ref[t]
            gates = (jnp.dot(emb, wih_e, preferred_element_type=jnp.float32)
                     + jnp.dot(rv, wih_r, preferred_element_type=jnp.float32)
                     + jnp.dot(h, whh, preferred_element_type=jnp.float32)
                     + bl)
            i_g = jax.nn.sigmoid(gates[:, 0:256])
            f_g = jax.nn.sigmoid(gates[:, 256:512])
            g_g = jnp.tanh(gates[:, 512:768])
            o_g = jax.nn.sigmoid(gates[:, 768:1024])
            c_new = f_g * c + i_g * g_g
            h_new = o_g * jnp.tanh(c_new)
            iface = jnp.dot(h_new, wif, preferred_element_type=jnp.float32) + bif
            rk = iface[:, 0:64]
            wk = iface[:, 64:128]
            wv = iface[:, 128:192]
            er = jax.nn.sigmoid(iface[:, 192:256])

            knr = normalize(rk)
            knw = normalize(wk)
            cols = cols_of(jnp.concatenate([knr, knw, wv, er], axis=1))
            knr_col = cols[:, 0:1]
            knw_col = cols[:, 1:2]
            wv_col = cols[:, 2:3]
            er_col = cols[:, 3:4]

            x3 = jnp.concatenate(
                [mem * mem, mem * knr_col, mem * knw_col], axis=1)
            red = jnp.dot(s3, x3, preferred_element_type=jnp.float32)
            inv = 1.0 / (jnp.sqrt(red[:, 0:_C]) + 1e-8)
            inv2 = jnp.concatenate([inv, inv], axis=0)
            beta_col = jnp.where(
                lax.broadcasted_iota(jnp.int32, (2 * _BH, 1), 0) < _BH,
                br, bw)
            scores = (jnp.concatenate([red[:, _C:2 * _C],
                                       red[:, 2 * _C:3 * _C]], axis=0)
                      * inv2 * beta_col)

            w8, ti8 = topk8(scores)
            maps = weight_map(w8, ti8)
            y = jnp.dot(mem, maps[0:_BH].T, preferred_element_type=jnp.float32)
            rvec_col = jnp.sum(y * s3t, axis=1, keepdims=True)
            rv_new = uncol(rvec_col)

            wmap = expand_rows(maps[_BH:2 * _BH])
            mem_ref[...] = mem + wmap * (wv_col - mem * er_col)
            h3_ref[t] = h_new
            return (h_new, c_new, rv_new)

        lax.fori_loop(0, _T, step, (h0, c0, r0))

    hmat = h3_ref[...].reshape(_T * _B, _HIDDEN).astype(jnp.bfloat16)
    out_ref[...] = (jnp.dot(hmat, wout_ref[...].astype(jnp.bfloat16),
                            preferred_element_type=jnp.float32)
                    + bout_ref[...])


def _gather_embeddings(embedding, idx):
    return jnp.take(embedding, idx, axis=0)


def kernel(input_seq, embedding, W_ih, W_hh, b_lstm, W_iface, b_iface,
           W_out, b_out, beta_read, beta_write):
    idx = input_seq.T.reshape(-1)
    emb_all = _gather_embeddings(embedding, idx).reshape(_T, _B, _EMBED)
    br = jnp.abs(beta_read) + 1e-6
    bw = jnp.abs(beta_write) + 1e-6
    betas = jnp.stack([br, bw]).reshape(1, 2).astype(jnp.float32)

    out = pl.pallas_call(
        _tc_kernel,
        grid=(_NV,),
        in_specs=[
            pl.BlockSpec((_T, _B, _EMBED), lambda v: (0, 0, 0)),
            pl.BlockSpec((_EMBED, 4 * _HIDDEN), lambda v: (0, 0)),
            pl.BlockSpec((_MEM_DIM, 4 * _HIDDEN), lambda v: (0, 0)),
            pl.BlockSpec((_HIDDEN, 4 * _HIDDEN), lambda v: (0, 0)),
            pl.BlockSpec((1, 4 * _HIDDEN), lambda v: (0, 0)),
            pl.BlockSpec((_HIDDEN, _HIDDEN), lambda v: (0, 0)),
            pl.BlockSpec((1, _HIDDEN), lambda v: (0, 0)),
            pl.BlockSpec(memory_space=pltpu.SMEM),
            pl.BlockSpec((_HIDDEN, _VBLK), lambda v: (0, v)),
            pl.BlockSpec((1, _VBLK), lambda v: (0, v)),
        ],
        out_specs=pl.BlockSpec((_T * _B, _VBLK), lambda v: (0, v)),
        out_shape=jax.ShapeDtypeStruct((_T * _B, _VOCAB), jnp.float32),
        scratch_shapes=[pltpu.VMEM((_T, _B, _HIDDEN), jnp.float32),
                        pltpu.VMEM((_ROWS, _C), jnp.float32)],
    )(emb_all, W_ih[:_EMBED], W_ih[_EMBED:], W_hh, b_lstm.reshape(1, -1),
      W_iface, b_iface.reshape(1, -1), betas, W_out, b_out.reshape(1, -1))

    return out.reshape(_T, _B, _VOCAB).transpose(1, 0, 2)
